# baseline (speedup 1.0000x reference)
import functools

import jax
import jax.numpy as jnp
from jax.experimental import pallas as pl
from jax.experimental.pallas import tpu as pltpu

_LOG1E4 = 9.210340371976184
_INV2PI = 0.15915494309189535

_C0 = 6.2797307080712255
_C1 = -41.13626070861352
_C2 = 78.32711789390086
_C3 = -57.11617448291767


def _pos_emb_kernel(x_ref, o_ref, inv_ref, ph_ref, *, ts: int, e: int):
    i = pl.program_id(0)

    @pl.when(i == 0)
    def _():
        ei = jax.lax.broadcasted_iota(jnp.int32, (1, e), 1)
        ef = ei.astype(jnp.float32)
        expo = (ef - jnp.mod(ef, 2.0)) * (1.0 / e)
        inv_ref[...] = jnp.exp(-_LOG1E4 * expo) * _INV2PI
        ph_ref[...] = jnp.where(ei % 2 == 0, 0.0, 0.25)

    pos = (i * ts + jax.lax.broadcasted_iota(jnp.int32, (ts, 1), 0)).astype(
        jnp.float32)
    y = pos * inv_ref[...] + ph_ref[...]
    r = y - jnp.floor(y + 0.5)
    r2 = r * r
    p = _C2 + r2 * _C3
    p = _C1 + r2 * p
    p = _C0 + r2 * p
    enc = r * p
    xv = x_ref[...]
    o_ref[...] = jnp.where(xv == 0.0, 0.0, enc[None, :, :])


def kernel(x):
    B, S, E = x.shape
    TS = 512
    grid = (S // TS,)
    return pl.pallas_call(
        functools.partial(_pos_emb_kernel, ts=TS, e=E),
        grid=grid,
        in_specs=[pl.BlockSpec((B, TS, E), lambda i: (0, i, 0))],
        out_specs=pl.BlockSpec((B, TS, E), lambda i: (0, i, 0)),
        out_shape=jax.ShapeDtypeStruct((B, S, E), jnp.float32),
        scratch_shapes=[
            pltpu.VMEM((1, E), jnp.float32),
            pltpu.VMEM((1, E), jnp.float32),
        ],
    )(x)
